# speedup vs baseline: 24.8269x; 24.8269x over previous
import jax
import jax.numpy as jnp
from jax.experimental import pallas as pl

H = 400
W = 400
K = 2000
PAD = 2

_HI_BITS = 0x3F800000


def _body(pred_ref, mask_ref, out_ref):
    x = pred_ref[0]
    m = mask_ref[0]
    p = m / (1.0 + jnp.exp(-x))
    keys = jax.lax.bitcast_convert_type(p, jnp.int32)

    def bis(_, carry):
        lo, hi = carry
        mid = jax.lax.shift_right_logical(lo + hi, 1)
        cnt = jnp.sum((keys >= mid).astype(jnp.int32))
        big = cnt >= K
        return (jnp.where(big, mid, lo), jnp.where(big, hi, mid))

    lo, _ = jax.lax.fori_loop(
        0, 30, bis, (jnp.int32(0), jnp.int32(_HI_BITS)))

    a = (keys >= lo).astype(jnp.float32)

    z = jnp.zeros_like(a)
    h = a
    for s in (1, 2):
        left = jnp.concatenate([a[:, s:], z[:, :s]], axis=1)
        right = jnp.concatenate([z[:, :s], a[:, :-s]], axis=1)
        h = jnp.maximum(h, jnp.maximum(left, right))
    v = h
    for s in (1, 2):
        up = jnp.concatenate([h[s:, :], z[:s, :]], axis=0)
        down = jnp.concatenate([z[:s, :], h[:-s, :]], axis=0)
        v = jnp.maximum(v, jnp.maximum(up, down))
    out_ref[0] = v > 0.0


def kernel(pred, pred_mask):
    b = pred.shape[0]
    predb = pred.reshape(b, H, W)
    return pl.pallas_call(
        _body,
        grid=(b,),
        in_specs=[
            pl.BlockSpec((1, H, W), lambda i: (i, 0, 0)),
            pl.BlockSpec((1, H, W), lambda i: (i, 0, 0)),
        ],
        out_specs=pl.BlockSpec((1, H, W), lambda i: (i, 0, 0)),
        out_shape=jax.ShapeDtypeStruct((b, H, W), jnp.bool_),
    )(predb, pred_mask)


# speedup vs baseline: 26.0933x; 1.0510x over previous
import jax
import jax.numpy as jnp
from jax.experimental import pallas as pl

H = 400
W = 400
K = 2000
_HI_BITS = 0x3F800000

_SUB_ROWS = 25
_RANK_LO = 192
_RANK_HI = 58


def _count(keys, t):
    return jnp.sum((keys >= t).astype(jnp.int32))


def _shift_up(a, z, s, axis):
    if axis == 0:
        return jnp.concatenate([a[s:, :], z[:s, :]], axis=0)
    return jnp.concatenate([a[:, s:], z[:, :s]], axis=1)


def _shift_down(a, z, s, axis):
    if axis == 0:
        return jnp.concatenate([z[:s, :], a[:-s, :]], axis=0)
    return jnp.concatenate([z[:, :s], a[:, :-s]], axis=1)


def _dilate5(a, z, axis):
    out = a
    for s in (1, 2):
        out = jnp.maximum(out, _shift_up(a, z, s, axis))
        out = jnp.maximum(out, _shift_down(a, z, s, axis))
    return out


def _body(pred_ref, mask_ref, out_ref):
    x = pred_ref[0]
    m = mask_ref[0]
    p = m / (1.0 + jnp.exp(-x))
    keys = jax.lax.bitcast_convert_type(p, jnp.int32)
    sub = keys[:_SUB_ROWS, :]

    def sub_search(rank):
        def bis(_, carry):
            lo, hi = carry
            mid = jax.lax.shift_right_logical(lo + hi, 1)
            big = _count(sub, mid) >= rank
            return (jnp.where(big, mid, lo), jnp.where(big, hi, mid))
        return jax.lax.fori_loop(
            0, 16, bis, (jnp.int32(0), jnp.int32(_HI_BITS)))

    lo0, _ = sub_search(_RANK_LO)
    _, hi0 = sub_search(_RANK_HI)

    cl = _count(keys, lo0)
    ch = _count(keys, hi0)
    lo = jnp.where(cl >= K, lo0, jnp.int32(0))
    hi = jnp.where(ch < K, hi0, jnp.int32(_HI_BITS))
    lo = jnp.where(ch == K, hi0, jnp.where(cl == K, lo0, lo))
    hi = jnp.where(ch == K, hi0 + 1, jnp.where(cl == K, lo0 + 1, hi))

    def cond(carry):
        lo, hi, it = carry
        return jnp.logical_and(hi - lo > 1, it < 34)

    def body(carry):
        lo, hi, it = carry
        mid = jax.lax.shift_right_logical(lo + hi, 1)
        cnt = _count(keys, mid)
        big = cnt >= K
        lo2 = jnp.where(big, mid, lo)
        hi2 = jnp.where(cnt == K, mid + 1, jnp.where(big, hi, mid))
        return (lo2, hi2, it + 1)

    thr, _, _ = jax.lax.while_loop(cond, body, (lo, hi, jnp.int32(0)))

    a = (keys >= thr).astype(jnp.float32)
    z = jnp.zeros_like(a)
    h = _dilate5(a, z, axis=1)
    v = _dilate5(h, z, axis=0)
    out_ref[0] = v > 0.0


def kernel(pred, pred_mask):
    b = pred.shape[0]
    predb = pred.reshape(b, H, W)
    return pl.pallas_call(
        _body,
        grid=(b,),
        in_specs=[
            pl.BlockSpec((1, H, W), lambda i: (i, 0, 0)),
            pl.BlockSpec((1, H, W), lambda i: (i, 0, 0)),
        ],
        out_specs=pl.BlockSpec((1, H, W), lambda i: (i, 0, 0)),
        out_shape=jax.ShapeDtypeStruct((b, H, W), jnp.bool_),
    )(predb, pred_mask)


# speedup vs baseline: 49.3089x; 1.8897x over previous
import jax
import jax.numpy as jnp
from jax.experimental import pallas as pl

H = 400
W = 400
K = 2000
BT = 8
_HI_BITS = 0x3F800000

_SUB_ROWS = 25
_RANK_LO = 192
_RANK_HI = 58


def _count(keys, t):
    return jnp.sum((keys >= t).astype(jnp.int32), axis=(1, 2), keepdims=True)


def _shift_up(a, z, s, axis):
    if axis == 1:
        return jnp.concatenate([a[:, s:, :], z[:, :s, :]], axis=1)
    return jnp.concatenate([a[:, :, s:], z[:, :, :s]], axis=2)


def _shift_down(a, z, s, axis):
    if axis == 1:
        return jnp.concatenate([z[:, :s, :], a[:, :-s, :]], axis=1)
    return jnp.concatenate([z[:, :, :s], a[:, :, :-s]], axis=2)


def _dilate5(a, z, axis):
    out = a
    for s in (1, 2):
        out = jnp.maximum(out, _shift_up(a, z, s, axis))
        out = jnp.maximum(out, _shift_down(a, z, s, axis))
    return out


def _body(pred_ref, mask_ref, out_ref):
    x = pred_ref[...]
    m = mask_ref[...]
    p = m / (1.0 + jnp.exp(-x))
    keys = jax.lax.bitcast_convert_type(p, jnp.int32)
    sub = keys[:, :_SUB_ROWS, :]

    def vfull(v):
        return jnp.full((BT, 1, 1), v, jnp.int32)

    def sub_search(rank):
        def bis(_, carry):
            lo, hi = carry
            mid = jax.lax.shift_right_logical(lo + hi, 1)
            big = _count(sub, mid) >= rank
            return (jnp.where(big, mid, lo), jnp.where(big, hi, mid))
        return jax.lax.fori_loop(
            0, 16, bis, (vfull(0), vfull(_HI_BITS)))

    lo0, _ = sub_search(_RANK_LO)
    _, hi0 = sub_search(_RANK_HI)

    cl = _count(keys, lo0)
    ch = _count(keys, hi0)
    lo = jnp.where(cl >= K, lo0, 0)
    hi = jnp.where(ch < K, hi0, _HI_BITS)
    lo = jnp.where(ch == K, hi0, jnp.where(cl == K, lo0, lo))
    hi = jnp.where(ch == K, hi0 + 1, jnp.where(cl == K, lo0 + 1, hi))

    def cond(carry):
        lo, hi, it = carry
        return jnp.logical_and(jnp.max(hi - lo) > 1, it < 40)

    def body(carry):
        lo, hi, it = carry
        mid = jax.lax.shift_right_logical(lo + hi, 1)
        cnt = _count(keys, mid)
        big = cnt >= K
        lo2 = jnp.where(big, mid, lo)
        hi2 = jnp.where(cnt == K, mid + 1, jnp.where(big, hi, mid))
        return (lo2, hi2, it + 1)

    thr, _, _ = jax.lax.while_loop(cond, body, (lo, hi, jnp.int32(0)))

    a = (keys >= thr).astype(jnp.float32)
    z = jnp.zeros_like(a)
    hmax = _dilate5(a, z, axis=2)
    v = _dilate5(hmax, z, axis=1)
    out_ref[...] = v > 0.0


def kernel(pred, pred_mask):
    b = pred.shape[0]
    predb = pred.reshape(b, H, W)
    return pl.pallas_call(
        _body,
        grid=(b // BT,),
        in_specs=[
            pl.BlockSpec((BT, H, W), lambda i: (i, 0, 0)),
            pl.BlockSpec((BT, H, W), lambda i: (i, 0, 0)),
        ],
        out_specs=pl.BlockSpec((BT, H, W), lambda i: (i, 0, 0)),
        out_shape=jax.ShapeDtypeStruct((b, H, W), jnp.bool_),
    )(predb, pred_mask)
